# Initial kernel scaffold; baseline (speedup 1.0000x reference)
import jax
import jax.numpy as jnp
from jax.experimental import pallas as pl


def kernel(x, edge_index, edge_weight, h, c, W_i, b_i, conv_i_W0, conv_i_W1, conv_i_b, W_f, b_f, conv_f_W0, conv_f_W1, conv_f_b, W_c, b_c, conv_c_W0, conv_c_W1, conv_c_b, W_o, b_o, conv_o_W0, conv_o_W1, conv_o_b, w_c_i, w_c_f, w_c_o, W_lin, b_lin):
    raise NotImplementedError("write your pallas kernel here")


# speedup vs baseline: 9.8085x; 9.8085x over previous
import functools

import jax
import jax.numpy as jnp
from jax import lax
from jax.experimental import pallas as pl
from jax.experimental.pallas import tpu as pltpu
from jax.experimental.pallas import tpu_sc as plsc

_N = 10000
_D = 128
_NC = 2
_NS = 16
_NW = _NC * _NS
_CHUNK = 128
_NP = 10240
_RPT = _NP // _NS


def _mesh():
    return plsc.VectorSubcoreMesh(
        core_axis_name="c", subcore_axis_name="s",
        num_cores=_NC, num_subcores=_NS)


def _deg_body(nchunk, row_hbm, w_hbm, deg_out, idx_v, w_v, zbuf, deg_sh, sem):
    cid = lax.axis_index("c")
    sid = lax.axis_index("s")
    wid = cid * _NS + sid
    zero = jnp.zeros((16,), jnp.float32)
    for k in range(_RPT // 16):
        zbuf[pl.ds(k * 16, 16)] = zero
    pltpu.sync_copy(zbuf, deg_sh.at[pl.ds(sid * _RPT, _RPT)])
    plsc.subcore_barrier()

    ept = nchunk * _CHUNK

    def body(j, carry):
        base = pl.multiple_of(wid * ept + j * _CHUNK, _CHUNK)
        pltpu.sync_copy(row_hbm.at[pl.ds(base, _CHUNK)], idx_v)
        pltpu.sync_copy(w_hbm.at[pl.ds(base, _CHUNK)], w_v)
        pltpu.sync_copy(w_v, deg_sh.at[idx_v], add=True)
        return carry

    lax.fori_loop(0, nchunk, body, 0)
    plsc.subcore_barrier()
    pltpu.sync_copy(deg_sh.at[pl.ds(sid * _RPT, _RPT)],
                    deg_out.at[cid, pl.ds(sid * _RPT, _RPT)])


def _make_deg(nchunk):
    return pl.kernel(
        functools.partial(_deg_body, nchunk),
        out_type=jax.ShapeDtypeStruct((_NC, _NP), jnp.float32),
        mesh=_mesh(),
        scratch_types=[
            pltpu.VMEM((_CHUNK,), jnp.int32),
            pltpu.VMEM((_CHUNK,), jnp.float32),
            pltpu.VMEM((_RPT,), jnp.float32),
            pltpu.VMEM_SHARED((_NP,), jnp.float32),
            pltpu.SemaphoreType.DMA,
        ],
        compiler_params=pltpu.CompilerParams(needs_layout_passes=False),
    )


def _spmm_body(nchunk, row_hbm, col_hbm, w_hbm, hp_hbm, u_out,
               ridx_v, cidx_v, w_v, rows_v, u_sh, sem):
    cid = lax.axis_index("c")
    sid = lax.axis_index("s")
    wid = cid * _NS + sid

    zero = jnp.zeros((16,), jnp.float32)

    def zbody(r, carry):
        for q in range(_D // 16):
            rows_v[r, pl.ds(q * 16, 16)] = zero
        return carry

    lax.fori_loop(0, _CHUNK, zbody, 0)
    for k in range(_RPT // _CHUNK):
        pltpu.sync_copy(rows_v, u_sh.at[pl.ds(sid * _RPT + k * _CHUNK, _CHUNK)])
    plsc.subcore_barrier()

    ept = nchunk * _CHUNK

    def body(j, carry):
        base = pl.multiple_of(wid * ept + j * _CHUNK, _CHUNK)
        pltpu.sync_copy(row_hbm.at[pl.ds(base, _CHUNK)], ridx_v)
        pltpu.sync_copy(col_hbm.at[pl.ds(base, _CHUNK)], cidx_v)
        pltpu.sync_copy(w_hbm.at[pl.ds(base, _CHUNK)], w_v)
        pltpu.async_copy(hp_hbm.at[ridx_v], rows_v, sem).wait()

        def scale(i, carry2):
            wb = plsc.load_gather(w_v, [jnp.zeros((16,), jnp.int32) + i])
            for q in range(_D // 16):
                rows_v[i, pl.ds(q * 16, 16)] = rows_v[i, pl.ds(q * 16, 16)] * wb
            return carry2

        lax.fori_loop(0, _CHUNK, scale, 0)
        pltpu.sync_copy(rows_v, u_sh.at[cidx_v], add=True)
        return carry

    lax.fori_loop(0, nchunk, body, 0)
    plsc.subcore_barrier()
    pltpu.sync_copy(u_sh.at[pl.ds(sid * _RPT, _RPT)],
                    u_out.at[cid, pl.ds(sid * _RPT, _RPT)])


def _make_spmm(nchunk):
    return pl.kernel(
        functools.partial(_spmm_body, nchunk),
        out_type=jax.ShapeDtypeStruct((_NC, _NP, _D), jnp.float32),
        mesh=_mesh(),
        scratch_types=[
            pltpu.VMEM((_CHUNK,), jnp.int32),
            pltpu.VMEM((_CHUNK,), jnp.int32),
            pltpu.VMEM((_CHUNK,), jnp.float32),
            pltpu.VMEM((_CHUNK, _D), jnp.float32),
            pltpu.VMEM_SHARED((_NP, _D), jnp.float32),
            pltpu.SemaphoreType.DMA,
        ],
        compiler_params=pltpu.CompilerParams(needs_layout_passes=False),
    )


def _disc_body(d0_ref, d1_ref, h_ref, hp_ref, dis_ref):
    deg = d0_ref[...] + d1_ref[...]
    pos = deg > 0
    dis = jnp.where(pos, lax.rsqrt(jnp.where(pos, deg, 1.0)), 0.0)
    dis_ref[...] = dis
    hp_ref[...] = dis * h_ref[...]


def _disc(d0, d1, h, blk, grid):
    return pl.pallas_call(
        _disc_body,
        grid=(grid,),
        in_specs=[
            pl.BlockSpec((blk, 1), lambda i: (i, 0)),
            pl.BlockSpec((blk, 1), lambda i: (i, 0)),
            pl.BlockSpec((blk, _D), lambda i: (i, 0)),
        ],
        out_specs=[
            pl.BlockSpec((blk, _D), lambda i: (i, 0)),
            pl.BlockSpec((blk, 1), lambda i: (i, 0)),
        ],
        out_shape=[
            jax.ShapeDtypeStruct(h.shape, jnp.float32),
            jax.ShapeDtypeStruct((h.shape[0], 1), jnp.float32),
        ],
    )(d0, d1, h)


def _dense_body(x_ref, h_ref, u0_ref, u1_ref, c_ref, dis_ref,
                wx_ref, w0_ref, w1_ref, bsum_ref,
                wci_ref, wcf_ref, wco_ref, wlin_ref, blin_ref,
                hout_ref, h0_ref, cn_ref):
    f32 = jnp.float32
    z = -dis_ref[...] * (u0_ref[...] + u1_ref[...])
    g = (jnp.dot(x_ref[...], wx_ref[...], preferred_element_type=f32)
         + jnp.dot(h_ref[...], w0_ref[...], preferred_element_type=f32)
         + jnp.dot(z, w1_ref[...], preferred_element_type=f32)
         + bsum_ref[0:1, :] + bsum_ref[1:2, :])
    cb = c_ref[...]
    gate_i = jax.nn.sigmoid(g[:, 0 * _D:1 * _D] + wci_ref[...] * cb)
    gate_f = jax.nn.sigmoid(g[:, 1 * _D:2 * _D] + wcf_ref[...] * cb)
    gate_t = jnp.tanh(g[:, 2 * _D:3 * _D])
    cn = gate_f * cb + gate_i * gate_t
    gate_o = jax.nn.sigmoid(g[:, 3 * _D:4 * _D] + wco_ref[...] * cn)
    h0 = gate_o * jnp.tanh(cn)
    cn_ref[...] = cn
    h0_ref[...] = h0
    hout_ref[...] = lax.dot_general(
        jnp.maximum(h0, 0.0), wlin_ref[...],
        (((1,), (1,)), ((), ())), preferred_element_type=f32) + blin_ref[...]


def _dense(x, h, u0, u1, c, dis, wx, w0, w1, bsum,
           wci, wcf, wco, wlin, blin, blk, grid):
    row_spec = pl.BlockSpec((blk, _D), lambda i: (i, 0))
    one_spec = pl.BlockSpec((blk, 1), lambda i: (i, 0))
    w_spec = pl.BlockSpec((_D, 4 * _D), lambda i: (0, 0))
    v_spec = pl.BlockSpec((1, _D), lambda i: (0, 0))
    return pl.pallas_call(
        _dense_body,
        grid=(grid,),
        in_specs=[
            row_spec, row_spec, row_spec, row_spec, row_spec, one_spec,
            w_spec, w_spec, w_spec,
            pl.BlockSpec((2, 4 * _D), lambda i: (0, 0)),
            v_spec, v_spec, v_spec,
            pl.BlockSpec((_D, _D), lambda i: (0, 0)),
            v_spec,
        ],
        out_specs=[row_spec, row_spec, row_spec],
        out_shape=[
            jax.ShapeDtypeStruct(x.shape, jnp.float32),
            jax.ShapeDtypeStruct(x.shape, jnp.float32),
            jax.ShapeDtypeStruct(x.shape, jnp.float32),
        ],
    )(x, h, u0, u1, c, dis, wx, w0, w1, bsum, wci, wcf, wco, wlin, blin)


def kernel(x, edge_index, edge_weight, h, c,
           W_i, b_i, conv_i_W0, conv_i_W1, conv_i_b,
           W_f, b_f, conv_f_W0, conv_f_W1, conv_f_b,
           W_c, b_c, conv_c_W0, conv_c_W1, conv_c_b,
           W_o, b_o, conv_o_W0, conv_o_W1, conv_o_b,
           w_c_i, w_c_f, w_c_o, W_lin, b_lin):
    n = x.shape[0]
    e = edge_weight.shape[0]
    assert n <= _NP and x.shape[1] == _D

    step = _NW * _CHUNK
    ep = ((e + step - 1) // step) * step
    pad = ep - e
    row = jnp.concatenate([edge_index[0], jnp.zeros((pad,), jnp.int32)])
    col = jnp.concatenate([edge_index[1], jnp.zeros((pad,), jnp.int32)])
    w = jnp.concatenate([edge_weight, jnp.zeros((pad,), jnp.float32)])
    nchunk = ep // step

    deg_part = _make_deg(nchunk)(row, w)
    d0 = deg_part[0, :n].reshape(n, 1)
    d1 = deg_part[1, :n].reshape(n, 1)

    blk, grid = 1000, n // 1000
    hp, dis = _disc(d0, d1, h, blk, grid)

    u_part = _make_spmm(nchunk)(row, col, w, hp)
    u0 = u_part[0, :n]
    u1 = u_part[1, :n]

    wx = jnp.concatenate([W_i, W_f, W_c, W_o], axis=1)
    w0 = jnp.concatenate([conv_i_W0, conv_f_W0, conv_c_W0, conv_o_W0], axis=1)
    w1 = jnp.concatenate([conv_i_W1, conv_f_W1, conv_c_W1, conv_o_W1], axis=1)
    bsum = jnp.stack([
        jnp.concatenate([b_i[0], b_f[0], b_c[0], b_o[0]]),
        jnp.concatenate([conv_i_b, conv_f_b, conv_c_b, conv_o_b]),
    ])

    return _dense(x, h, u0, u1, c, dis, wx, w0, w1, bsum,
                  w_c_i, w_c_f, w_c_o, W_lin, b_lin.reshape(1, _D), blk, grid)


# speedup vs baseline: 10.0950x; 1.0292x over previous
import functools

import jax
import jax.numpy as jnp
from jax import lax
from jax.experimental import pallas as pl
from jax.experimental.pallas import tpu as pltpu
from jax.experimental.pallas import tpu_sc as plsc

_N = 10000
_D = 128
_NC = 2
_NS = 16
_NW = _NC * _NS
_CHUNK = 128
_NP = 10240
_RPT = _NP // _NS


def _mesh():
    return plsc.VectorSubcoreMesh(
        core_axis_name="c", subcore_axis_name="s",
        num_cores=_NC, num_subcores=_NS)


def _deg_body(nchunk, row_hbm, w_hbm, deg_out,
              ridx0, ridx1, w0, w1, zbuf, deg_sh, isem0, isem1):
    cid = lax.axis_index("c")
    sid = lax.axis_index("s")
    wid = cid * _NS + sid
    ridx = (ridx0, ridx1)
    wv = (w0, w1)
    isem = (isem0, isem1)
    zero = jnp.zeros((16,), jnp.float32)
    for k in range(_RPT // 16):
        zbuf[pl.ds(k * 16, 16)] = zero
    pltpu.sync_copy(zbuf, deg_sh.at[pl.ds(sid * _RPT, _RPT)])

    ept = nchunk * _CHUNK
    plsc.subcore_barrier()

    def body(j, carry):
        base = pl.multiple_of(wid * ept + j * _CHUNK, _CHUNK)
        pltpu.sync_copy(row_hbm.at[pl.ds(base, _CHUNK)], ridx[0])
        pltpu.sync_copy(w_hbm.at[pl.ds(base, _CHUNK)], wv[0])
        pltpu.sync_copy(wv[0], deg_sh.at[ridx[0]], add=True)
        return carry

    lax.fori_loop(0, nchunk, body, 0)
    plsc.subcore_barrier()
    pltpu.sync_copy(deg_sh.at[pl.ds(sid * _RPT, _RPT)],
                    deg_out.at[cid, pl.ds(sid * _RPT, _RPT)])


def _make_deg(nchunk):
    return pl.kernel(
        functools.partial(_deg_body, nchunk),
        out_type=jax.ShapeDtypeStruct((_NC, _NP), jnp.float32),
        mesh=_mesh(),
        scratch_types=[
            pltpu.VMEM((_CHUNK,), jnp.int32),
            pltpu.VMEM((_CHUNK,), jnp.int32),
            pltpu.VMEM((_CHUNK,), jnp.float32),
            pltpu.VMEM((_CHUNK,), jnp.float32),
            pltpu.VMEM((_RPT,), jnp.float32),
            pltpu.VMEM_SHARED((_NP,), jnp.float32),
            pltpu.SemaphoreType.DMA,
            pltpu.SemaphoreType.DMA,
        ],
        compiler_params=pltpu.CompilerParams(needs_layout_passes=False),
    )


def _spmm_body(nchunk, row_hbm, col_hbm, w_hbm, hp_hbm, u_out,
               ridx0, ridx1, cidx0, cidx1, w0, w1, rows0, rows1, u_sh,
               isem0, isem1, gsem0, gsem1):
    cid = lax.axis_index("c")
    sid = lax.axis_index("s")
    wid = cid * _NS + sid
    ridx = (ridx0, ridx1)
    cidx = (cidx0, cidx1)
    wv = (w0, w1)
    rows = (rows0, rows1)
    isem = (isem0, isem1)
    gsem = (gsem0, gsem1)

    zero = jnp.zeros((16,), jnp.float32)

    def zbody(r, carry):
        for q in range(_D // 16):
            rows0[r, pl.ds(q * 16, 16)] = zero
        return carry

    lax.fori_loop(0, _CHUNK, zbody, 0)
    for k in range(_RPT // _CHUNK):
        pltpu.sync_copy(rows0, u_sh.at[pl.ds(sid * _RPT + k * _CHUNK,
                                             _CHUNK)])

    ept = nchunk * _CHUNK

    def idx_issue(k, b):
        base = pl.multiple_of(wid * ept + k * _CHUNK, _CHUNK)
        pltpu.async_copy(row_hbm.at[pl.ds(base, _CHUNK)], ridx[b], isem[b])
        pltpu.async_copy(col_hbm.at[pl.ds(base, _CHUNK)], cidx[b], isem[b])
        pltpu.async_copy(w_hbm.at[pl.ds(base, _CHUNK)], wv[b], isem[b])

    def idx_wait(b):
        pltpu.make_async_copy(row_hbm.at[pl.ds(0, _CHUNK)], ridx[b],
                              isem[b]).wait()
        pltpu.make_async_copy(col_hbm.at[pl.ds(0, _CHUNK)], cidx[b],
                              isem[b]).wait()
        pltpu.make_async_copy(w_hbm.at[pl.ds(0, _CHUNK)], wv[b],
                              isem[b]).wait()

    def idx_load(k, b):
        base = pl.multiple_of(wid * ept + k * _CHUNK, _CHUNK)
        pltpu.sync_copy(row_hbm.at[pl.ds(base, _CHUNK)], ridx[b])
        pltpu.sync_copy(col_hbm.at[pl.ds(base, _CHUNK)], cidx[b])
        pltpu.sync_copy(w_hbm.at[pl.ds(base, _CHUNK)], wv[b])

    def scale_chunk(b):
        def scale(i, carry2):
            for u in range(4):
                i4 = i * 4 + u
                wb = plsc.load_gather(
                    wv[b], [jnp.zeros((16,), jnp.int32) + i4])
                for q in range(_D // 16):
                    rows[b][i4, pl.ds(q * 16, 16)] = (
                        rows[b][i4, pl.ds(q * 16, 16)] * wb)
            return carry2

        lax.fori_loop(0, _CHUNK // 4, scale, 0)

    plsc.subcore_barrier()

    def body(k, carry):
        idx_load(k, 0)
        pltpu.async_copy(hp_hbm.at[ridx[0]], rows[0], gsem[0]).wait()
        scale_chunk(0)
        pltpu.sync_copy(rows[0], u_sh.at[cidx[0]], add=True)
        return carry

    lax.fori_loop(0, nchunk, body, 0)
    plsc.subcore_barrier()
    pltpu.sync_copy(u_sh.at[pl.ds(sid * _RPT, _RPT)],
                    u_out.at[cid, pl.ds(sid * _RPT, _RPT)])


def _make_spmm(nchunk):
    return pl.kernel(
        functools.partial(_spmm_body, nchunk),
        out_type=jax.ShapeDtypeStruct((_NC, _NP, _D), jnp.float32),
        mesh=_mesh(),
        scratch_types=[
            pltpu.VMEM((_CHUNK,), jnp.int32),
            pltpu.VMEM((_CHUNK,), jnp.int32),
            pltpu.VMEM((_CHUNK,), jnp.int32),
            pltpu.VMEM((_CHUNK,), jnp.int32),
            pltpu.VMEM((_CHUNK,), jnp.float32),
            pltpu.VMEM((_CHUNK,), jnp.float32),
            pltpu.VMEM((_CHUNK, _D), jnp.float32),
            pltpu.VMEM((_CHUNK, _D), jnp.float32),
            pltpu.VMEM_SHARED((_NP, _D), jnp.float32),
            pltpu.SemaphoreType.DMA,
            pltpu.SemaphoreType.DMA,
            pltpu.SemaphoreType.DMA,
            pltpu.SemaphoreType.DMA,
        ],
        compiler_params=pltpu.CompilerParams(needs_layout_passes=False),
    )


def _disc_body(d0_ref, d1_ref, h_ref, hp_ref, dis_ref):
    deg = d0_ref[...] + d1_ref[...]
    pos = deg > 0
    dis = jnp.where(pos, lax.rsqrt(jnp.where(pos, deg, 1.0)), 0.0)
    dis_ref[...] = dis
    hp_ref[...] = dis * h_ref[...]


def _disc(d0, d1, h, blk, grid):
    return pl.pallas_call(
        _disc_body,
        grid=(grid,),
        in_specs=[
            pl.BlockSpec((blk, 1), lambda i: (i, 0)),
            pl.BlockSpec((blk, 1), lambda i: (i, 0)),
            pl.BlockSpec((blk, _D), lambda i: (i, 0)),
        ],
        out_specs=[
            pl.BlockSpec((blk, _D), lambda i: (i, 0)),
            pl.BlockSpec((blk, 1), lambda i: (i, 0)),
        ],
        out_shape=[
            jax.ShapeDtypeStruct(h.shape, jnp.float32),
            jax.ShapeDtypeStruct((h.shape[0], 1), jnp.float32),
        ],
    )(d0, d1, h)


def _dense_body(x_ref, h_ref, u0_ref, u1_ref, c_ref, dis_ref,
                wx_ref, w0_ref, w1_ref, bsum_ref,
                wci_ref, wcf_ref, wco_ref, wlin_ref, blin_ref,
                hout_ref, h0_ref, cn_ref):
    f32 = jnp.float32
    z = -dis_ref[...] * (u0_ref[...] + u1_ref[...])
    g = (jnp.dot(x_ref[...], wx_ref[...], preferred_element_type=f32)
         + jnp.dot(h_ref[...], w0_ref[...], preferred_element_type=f32)
         + jnp.dot(z, w1_ref[...], preferred_element_type=f32)
         + bsum_ref[0:1, :] + bsum_ref[1:2, :])
    cb = c_ref[...]
    gate_i = jax.nn.sigmoid(g[:, 0 * _D:1 * _D] + wci_ref[...] * cb)
    gate_f = jax.nn.sigmoid(g[:, 1 * _D:2 * _D] + wcf_ref[...] * cb)
    gate_t = jnp.tanh(g[:, 2 * _D:3 * _D])
    cn = gate_f * cb + gate_i * gate_t
    gate_o = jax.nn.sigmoid(g[:, 3 * _D:4 * _D] + wco_ref[...] * cn)
    h0 = gate_o * jnp.tanh(cn)
    cn_ref[...] = cn
    h0_ref[...] = h0
    hout_ref[...] = lax.dot_general(
        jnp.maximum(h0, 0.0), wlin_ref[...],
        (((1,), (1,)), ((), ())), preferred_element_type=f32) + blin_ref[...]


def _dense(x, h, u0, u1, c, dis, wx, w0, w1, bsum,
           wci, wcf, wco, wlin, blin, blk, grid):
    row_spec = pl.BlockSpec((blk, _D), lambda i: (i, 0))
    one_spec = pl.BlockSpec((blk, 1), lambda i: (i, 0))
    w_spec = pl.BlockSpec((_D, 4 * _D), lambda i: (0, 0))
    v_spec = pl.BlockSpec((1, _D), lambda i: (0, 0))
    return pl.pallas_call(
        _dense_body,
        grid=(grid,),
        in_specs=[
            row_spec, row_spec, row_spec, row_spec, row_spec, one_spec,
            w_spec, w_spec, w_spec,
            pl.BlockSpec((2, 4 * _D), lambda i: (0, 0)),
            v_spec, v_spec, v_spec,
            pl.BlockSpec((_D, _D), lambda i: (0, 0)),
            v_spec,
        ],
        out_specs=[row_spec, row_spec, row_spec],
        out_shape=[
            jax.ShapeDtypeStruct(x.shape, jnp.float32),
            jax.ShapeDtypeStruct(x.shape, jnp.float32),
            jax.ShapeDtypeStruct(x.shape, jnp.float32),
        ],
    )(x, h, u0, u1, c, dis, wx, w0, w1, bsum, wci, wcf, wco, wlin, blin)


def kernel(x, edge_index, edge_weight, h, c,
           W_i, b_i, conv_i_W0, conv_i_W1, conv_i_b,
           W_f, b_f, conv_f_W0, conv_f_W1, conv_f_b,
           W_c, b_c, conv_c_W0, conv_c_W1, conv_c_b,
           W_o, b_o, conv_o_W0, conv_o_W1, conv_o_b,
           w_c_i, w_c_f, w_c_o, W_lin, b_lin):
    n = x.shape[0]
    e = edge_weight.shape[0]
    assert n <= _NP and x.shape[1] == _D

    step = _NW * _CHUNK
    ep = ((e + step - 1) // step) * step
    pad = ep - e
    row = jnp.concatenate([edge_index[0], jnp.zeros((pad,), jnp.int32)])
    col = jnp.concatenate([edge_index[1], jnp.zeros((pad,), jnp.int32)])
    w = jnp.concatenate([edge_weight, jnp.zeros((pad,), jnp.float32)])
    nchunk = ep // step

    deg_part = _make_deg(nchunk)(row, w)
    d0 = deg_part[0, :n].reshape(n, 1)
    d1 = deg_part[1, :n].reshape(n, 1)

    blk, grid = 1000, n // 1000
    hp, dis = _disc(d0, d1, h, blk, grid)

    u_part = _make_spmm(nchunk)(row, col, w, hp)
    u0 = u_part[0, :n]
    u1 = u_part[1, :n]

    wx = jnp.concatenate([W_i, W_f, W_c, W_o], axis=1)
    w0 = jnp.concatenate([conv_i_W0, conv_f_W0, conv_c_W0, conv_o_W0], axis=1)
    w1 = jnp.concatenate([conv_i_W1, conv_f_W1, conv_c_W1, conv_o_W1], axis=1)
    bsum = jnp.stack([
        jnp.concatenate([b_i[0], b_f[0], b_c[0], b_o[0]]),
        jnp.concatenate([conv_i_b, conv_f_b, conv_c_b, conv_o_b]),
    ])

    return _dense(x, h, u0, u1, c, dis, wx, w0, w1, bsum,
                  w_c_i, w_c_f, w_c_o, W_lin, b_lin.reshape(1, _D), blk, grid)
